# baseline (speedup 1.0000x reference)
import functools

import jax
import jax.numpy as jnp
from jax.experimental import pallas as pl
from jax.experimental.pallas import tpu as pltpu

_TOPK = 32
_KB = 256
_DEPTH = 5
_LANES = 128
_EPS = 1e-12
_NEG = float("-inf")


def _norm_rows_body(x_ref, o_ref):
    x = x_ref[...]
    n = jnp.sqrt(jnp.sum(x * x, axis=1, keepdims=True))
    o_ref[...] = x / jnp.maximum(n, _EPS)


def _full_extract(nq, kb, s_ref, slot, vals_ref, idx_ref):
    iota = jax.lax.broadcasted_iota(jnp.int32, (kb, nq), 1)
    col = jax.lax.broadcasted_iota(jnp.int32, (kb, _TOPK), 1)

    def step(i, carry):
        vals, idxs = carry
        s = s_ref[slot]
        m = jnp.max(s, axis=1, keepdims=True)
        idx = jnp.min(jnp.where(s == m, iota, nq), axis=1, keepdims=True)
        s_ref[slot] = jnp.where(iota == idx, _NEG, s)
        vals = jnp.where(col == i, m, vals)
        idxs = jnp.where(col == i, idx, idxs)
        return vals, idxs

    vals0 = jnp.zeros((kb, _TOPK), jnp.float32)
    idxs0 = jnp.zeros((kb, _TOPK), jnp.int32)
    vals, idxs = jax.lax.fori_loop(0, _TOPK, step, (vals0, idxs0))
    vals_ref[...] = vals
    idx_ref[...] = idxs


def _treemax(xs):
    while len(xs) > 1:
        xs = [jnp.maximum(a, b) for a, b in zip(xs[::2], xs[1::2])] + (
            [xs[-1]] if len(xs) % 2 else [])
    return xs[0]


def _treemin(xs):
    while len(xs) > 1:
        xs = [jnp.minimum(a, b) for a, b in zip(xs[::2], xs[1::2])] + (
            [xs[-1]] if len(xs) % 2 else [])
    return xs[0]


def _extract_phases(nq, kb, depth, s_ref, slot, kn_ref, q, vals_ref, idx_ref,
                    vc_ref, ic_ref):
    nm = nq // _LANES
    lane = jax.lax.broadcasted_iota(jnp.int32, (kb, _LANES), 1)
    big = jnp.float32(nm)

    v_parts, i_parts = [], []
    for d in range(depth):
        chunks = [s_ref[slot, :, j * _LANES:(j + 1) * _LANES]
                  for j in range(nm)]
        g = _treemax(chunks)
        mj = _treemin([jnp.where(c == g, jnp.float32(j), big)
                       for j, c in enumerate(chunks)])
        v_parts.append(g)
        i_parts.append(mj.astype(jnp.int32) * _LANES + lane)
        if d + 1 < depth:
            for j in range(nm):
                s_ref[slot, :, j * _LANES:(j + 1) * _LANES] = jnp.where(
                    mj == jnp.float32(j), _NEG, chunks[j])
    vc_ref[...] = jnp.concatenate(v_parts, axis=1)
    ic_ref[...] = jnp.concatenate(i_parts, axis=1)

    col = jax.lax.broadcasted_iota(jnp.int32, (kb, _TOPK), 1)

    def step(i, carry):
        vals, idxs, _ = carry
        v = vc_ref[...]
        m = jnp.max(v, axis=1, keepdims=True)
        io = jnp.min(jnp.where(v == m, ic_ref[...], nq), axis=1, keepdims=True)
        vc_ref[...] = jnp.where(ic_ref[...] == io, _NEG, v)
        vals = jnp.where(col == i, m, vals)
        idxs = jnp.where(col == i, io, idxs)
        return vals, idxs, m

    vals0 = jnp.zeros((kb, _TOPK), jnp.float32)
    idxs0 = jnp.zeros((kb, _TOPK), jnp.int32)
    m0 = jnp.zeros((kb, 1), jnp.float32)
    vals, idxs, v_last = jax.lax.fori_loop(0, _TOPK, step, (vals0, idxs0, m0),
                                           unroll=4)
    vals_ref[...] = vals
    idx_ref[...] = idxs

    if depth < nm:
        unsafe = jnp.any(v_parts[depth - 1] >= v_last)

        @pl.when(unsafe)
        def _fallback():
            s_ref[slot] = jax.lax.dot_general(
                kn_ref[slot], q, (((1,), (1,)), ((), ())),
                preferred_element_type=jnp.float32)
            _full_extract(nq, kb, s_ref, slot, vals_ref, idx_ref)


def _topk_body(nq, kb, depth, nb, keys_ref, q_ref, vals_ref, idx_ref,
               s2_ref, kn2_ref, vc_ref, ic_ref):
    i = pl.program_id(0)
    slot = jax.lax.rem(i, 2)
    prev = jax.lax.rem(i + 1, 2)
    q = q_ref[...]

    @pl.when(i < nb)
    def _matmul():
        kmat = keys_ref[...]
        n = jnp.sqrt(jnp.sum(kmat * kmat, axis=1, keepdims=True))
        kmat = kmat / jnp.maximum(n, _EPS)
        kn2_ref[slot] = kmat
        s2_ref[slot] = jax.lax.dot_general(
            kmat, q, (((1,), (1,)), ((), ())),
            preferred_element_type=jnp.float32)

    @pl.when(i > 0)
    def _extract():
        _extract_phases(nq, kb, depth, s2_ref, prev, kn2_ref, q,
                        vals_ref, idx_ref, vc_ref, ic_ref)


def kernel(query, keys, k):
    del k
    nq, d = query.shape
    nk = keys.shape[0]
    nm = nq // _LANES
    depth = min(_DEPTH, nm)

    qblk = min(nq, 512)
    qn = pl.pallas_call(
        _norm_rows_body,
        out_shape=jax.ShapeDtypeStruct((nq, d), jnp.float32),
        grid=(pl.cdiv(nq, qblk),),
        in_specs=[pl.BlockSpec((qblk, d), lambda j: (j, 0))],
        out_specs=pl.BlockSpec((qblk, d), lambda j: (j, 0)),
    )(query)

    nblocks = pl.cdiv(nk, _KB)
    vals, idxs = pl.pallas_call(
        functools.partial(_topk_body, nq, _KB, depth, nblocks),
        out_shape=(jax.ShapeDtypeStruct((nk, _TOPK), jnp.float32),
                   jax.ShapeDtypeStruct((nk, _TOPK), jnp.int32)),
        grid=(nblocks + 1,),
        in_specs=[
            pl.BlockSpec((_KB, d),
                         lambda i: (jnp.minimum(i, nblocks - 1), 0)),
            pl.BlockSpec((nq, d), lambda i: (0, 0)),
        ],
        out_specs=(
            pl.BlockSpec((_KB, _TOPK), lambda i: (jnp.maximum(i - 1, 0), 0)),
            pl.BlockSpec((_KB, _TOPK), lambda i: (jnp.maximum(i - 1, 0), 0)),
        ),
        scratch_shapes=[pltpu.VMEM((2, _KB, nq), jnp.float32),
                        pltpu.VMEM((2, _KB, d), jnp.float32),
                        pltpu.VMEM((_KB, depth * _LANES), jnp.float32),
                        pltpu.VMEM((_KB, depth * _LANES), jnp.int32)],
    )(keys, qn)
    return vals, idxs


# speedup vs baseline: 1.0878x; 1.0878x over previous
import functools

import jax
import jax.numpy as jnp
from jax.experimental import pallas as pl
from jax.experimental.pallas import tpu as pltpu

_TOPK = 32
_KB = 256
_DEPTH = 5
_LANES = 128
_EPS = 1e-12
_NEG = float("-inf")


def _norm_rows_body(x_ref, o_ref):
    x = x_ref[...]
    n = jnp.sqrt(jnp.sum(x * x, axis=1, keepdims=True))
    o_ref[...] = x / jnp.maximum(n, _EPS)


def _full_extract(nq, kb, s_ref, vals_ref, idx_ref):
    iota = jax.lax.broadcasted_iota(jnp.int32, (kb, nq), 1)
    col = jax.lax.broadcasted_iota(jnp.int32, (kb, _TOPK), 1)

    def step(i, carry):
        vals, idxs = carry
        s = s_ref[...]
        m = jnp.max(s, axis=1, keepdims=True)
        idx = jnp.min(jnp.where(s == m, iota, nq), axis=1, keepdims=True)
        s_ref[...] = jnp.where(iota == idx, _NEG, s)
        vals = jnp.where(col == i, m, vals)
        idxs = jnp.where(col == i, idx, idxs)
        return vals, idxs

    vals0 = jnp.zeros((kb, _TOPK), jnp.float32)
    idxs0 = jnp.zeros((kb, _TOPK), jnp.int32)
    vals, idxs = jax.lax.fori_loop(0, _TOPK, step, (vals0, idxs0))
    vals_ref[...] = vals
    idx_ref[...] = idxs


def _treemax(xs):
    while len(xs) > 1:
        xs = [jnp.maximum(a, b) for a, b in zip(xs[::2], xs[1::2])] + (
            [xs[-1]] if len(xs) % 2 else [])
    return xs[0]


def _treemin(xs):
    while len(xs) > 1:
        xs = [jnp.minimum(a, b) for a, b in zip(xs[::2], xs[1::2])] + (
            [xs[-1]] if len(xs) % 2 else [])
    return xs[0]


def _topk_body(nq, kb, depth, keys_ref, q_ref, vals_ref, idx_ref,
               s_ref, vc_ref, ic_ref):
    nm = nq // _LANES
    kmat = keys_ref[...]
    n = jnp.sqrt(jnp.sum(kmat * kmat, axis=1, keepdims=True))
    kmat = kmat / jnp.maximum(n, _EPS)
    q = q_ref[...]
    dot = lambda: jax.lax.dot_general(
        kmat, q, (((1,), (1,)), ((), ())), preferred_element_type=jnp.float32)
    s_ref[...] = dot()

    lane = jax.lax.broadcasted_iota(jnp.int32, (kb, _LANES), 1)
    big = jnp.float32(nm)

    chunks = [s_ref[:, j * _LANES:(j + 1) * _LANES] for j in range(nm)]
    v_parts, i_parts = [], []
    for d in range(depth):
        g = _treemax(chunks)
        mj = _treemin([jnp.where(c == g, jnp.float32(j), big)
                       for j, c in enumerate(chunks)])
        v_parts.append(g)
        i_parts.append(mj.astype(jnp.int32) * _LANES + lane)
        if d + 1 < depth:
            chunks = [jnp.where(mj == jnp.float32(j), _NEG, c)
                      for j, c in enumerate(chunks)]
    vc_ref[...] = jnp.concatenate(v_parts, axis=1)
    ic_ref[...] = jnp.concatenate(i_parts, axis=1)

    col = jax.lax.broadcasted_iota(jnp.int32, (kb, _TOPK), 1)

    def step(i, carry):
        vals, idxs, last = carry
        v = vc_ref[...]
        ic = ic_ref[...]
        for t in range(4):
            m = jnp.max(v, axis=1, keepdims=True)
            io = jnp.min(jnp.where(v == m, ic, nq), axis=1, keepdims=True)
            v = jnp.where(ic == io, _NEG, v)
            j = 4 * i + t
            vals = jnp.where(col == j, m, vals)
            idxs = jnp.where(col == j, io, idxs)
            last = m
        vc_ref[...] = v
        return vals, idxs, last

    vals0 = jnp.zeros((kb, _TOPK), jnp.float32)
    idxs0 = jnp.zeros((kb, _TOPK), jnp.int32)
    m0 = jnp.zeros((kb, 1), jnp.float32)
    vals, idxs, v_last = jax.lax.fori_loop(0, _TOPK // 4, step,
                                           (vals0, idxs0, m0), unroll=2)
    vals_ref[...] = vals
    idx_ref[...] = idxs

    if depth < nm:
        unsafe = jnp.any(v_parts[depth - 1] >= v_last)

        @pl.when(unsafe)
        def _fallback():
            s_ref[...] = dot()
            _full_extract(nq, kb, s_ref, vals_ref, idx_ref)


def kernel(query, keys, k):
    del k
    nq, d = query.shape
    nk = keys.shape[0]
    nm = nq // _LANES
    depth = min(_DEPTH, nm)

    qblk = min(nq, 512)
    qn = pl.pallas_call(
        _norm_rows_body,
        out_shape=jax.ShapeDtypeStruct((nq, d), jnp.float32),
        grid=(pl.cdiv(nq, qblk),),
        in_specs=[pl.BlockSpec((qblk, d), lambda j: (j, 0))],
        out_specs=pl.BlockSpec((qblk, d), lambda j: (j, 0)),
    )(query)

    nblocks = pl.cdiv(nk, _KB)
    vals, idxs = pl.pallas_call(
        functools.partial(_topk_body, nq, _KB, depth),
        out_shape=(jax.ShapeDtypeStruct((nk, _TOPK), jnp.float32),
                   jax.ShapeDtypeStruct((nk, _TOPK), jnp.int32)),
        grid=(nblocks,),
        in_specs=[pl.BlockSpec((_KB, d), lambda i: (i, 0)),
                  pl.BlockSpec((nq, d), lambda i: (0, 0))],
        out_specs=(pl.BlockSpec((_KB, _TOPK), lambda i: (i, 0)),
                   pl.BlockSpec((_KB, _TOPK), lambda i: (i, 0))),
        scratch_shapes=[pltpu.VMEM((_KB, nq), jnp.float32),
                        pltpu.VMEM((_KB, depth * _LANES), jnp.float32),
                        pltpu.VMEM((_KB, depth * _LANES), jnp.int32)],
    )(keys, qn)
    return vals, idxs
